# Initial kernel scaffold; baseline (speedup 1.0000x reference)
import jax
import jax.numpy as jnp
from jax.experimental import pallas as pl


def kernel(x, edge_index, batch, y, W_gcn, b_gcn, W1, as1, ad1, b1g, W2, as2, ad2, b2g, Wl1, bl1, Wl2, bl2):
    raise NotImplementedError("write your pallas kernel here")


# speedup vs baseline: 2.8126x; 2.8126x over previous
from jax.experimental import pallas as pl
import jax, jax.numpy as jnp
import numpy as np
N, E, G, HEADS = 10000, 320000, 64, 8
LAT = [32, 32, 32]
C2 = 8192
C1 = 114688
S1CAP = C2 + G


def kernel(x, edge_index, batch, y, W_gcn, b_gcn, W1, as1, ad1, b1g, W2, as2, ad2, b2g, Wl1, bl1, Wl2, bl2):
    src, dst = edge_index[0], edge_index[1]
    loop = jnp.arange(N, dtype=src.dtype)
    s = jnp.concatenate([src, loop])
    d = jnp.concatenate([dst, loop])
    xw = x @ W_gcn
    deg = jax.ops.segment_sum(jnp.ones(s.shape[0], jnp.float32), d, num_segments=N)
    dinv = jnp.where(deg > 0, deg ** -0.5, 0.0)
    norm = dinv[s] * dinv[d]
    x1 = jax.nn.elu(jax.ops.segment_sum(xw[s] * norm[:, None], d, num_segments=N) + b_gcn)
    idx = jnp.searchsorted(batch, jnp.arange(G, dtype=batch.dtype)).astype(jnp.int32)
    mark0 = jnp.zeros(N, bool).at[idx].set(True)
    sel2 = mark0[dst]
    e2 = jnp.nonzero(sel2, size=C2, fill_value=E)[0]
    v2 = e2 < E
    e2c = jnp.minimum(e2, E - 1)
    s2 = jnp.where(v2, src[e2c], 0)
    d2 = jnp.where(v2, dst[e2c], 0)
    s2 = jnp.concatenate([s2, idx]); d2 = jnp.concatenate([d2, idx])
    v2 = jnp.concatenate([v2, jnp.ones(G, bool)])
    mark1 = mark0.at[jnp.where(v2, s2, N)].set(True, mode="drop")
    sel1 = mark1[dst]
    e1 = jnp.nonzero(sel1, size=C1, fill_value=E)[0]
    v1 = e1 < E
    e1c = jnp.minimum(e1, E - 1)
    s1 = jnp.where(v1, src[e1c], 0)
    d1 = jnp.where(v1, dst[e1c], 0)
    n1 = jnp.nonzero(mark1, size=S1CAP, fill_value=N)[0]
    nv1 = n1 < N
    n1c = jnp.where(nv1, n1, 0).astype(jnp.int32)
    s1 = jnp.concatenate([s1, n1c]); d1 = jnp.concatenate([d1, n1c])
    v1 = jnp.concatenate([v1, nv1])

    def gat(xin, W, a_s_w, a_d_w, b, se, de, ve, out_ch):
        h = (xin @ W).reshape(N, HEADS, out_ch)
        a_s = (h * a_s_w[None]).sum(-1)
        a_d = (h * a_d_w[None]).sum(-1)
        e_att = jax.nn.leaky_relu(a_s[se] + a_d[de], 0.2)
        M = jnp.max(jnp.where(ve[:, None], e_att, -jnp.inf), axis=0)
        ex = jnp.exp(e_att - M[None, :]) * ve[:, None]
        den = jax.ops.segment_sum(ex, de, num_segments=N)
        alpha = ex / (den[de] + 1e-16)
        agg = jax.ops.segment_sum(h[se] * alpha[:, :, None], de, num_segments=N)
        return jax.nn.elu(agg.reshape(N, HEADS * out_ch) + b)

    x2 = gat(x1, W1, as1, ad1, b1g, s1, d1, v1, LAT[1])
    x3 = gat(x2, W2, as2, ad2, b2g, s2, d2, v2, LAT[2])
    cat = jnp.concatenate([x1[idx], x2[idx], x3[idx]], axis=1)
    feat = cat @ Wl1 + bl1
    h = jax.nn.elu(feat)
    logits = h @ Wl2 + bl2
    logp = jax.nn.log_softmax(logits, axis=1)
    prob = jax.nn.softmax(logits, axis=1)
    loss = -jnp.mean(logp[jnp.arange(G), y])
    pred = jnp.argmax(logp, axis=1)
    return (logp, loss, pred, prob, feat)


# speedup vs baseline: 4.5036x; 1.6012x over previous
import functools
import jax, jax.numpy as jnp
from jax import lax
from jax.experimental import pallas as pl
from jax.experimental.pallas import tpu as pltpu
from jax.experimental.pallas import tpu_sc as plsc

N, E, G, HEADS = 10000, 320000, 64, 8
LAT = [32, 32, 32]
NP = 10112
C2 = 4096
C1 = 98304
S1CAP = C2 - G + G
SNP1 = 4224
SNP2 = 128
NC, NS = 2, 16
NW = NC * NS
EPW = E // NW
ROWS_T = NP // NS

_mesh = functools.partial(plsc.VectorSubcoreMesh,
                          core_axis_name="c", subcore_axis_name="s")


def _worker():
    c = lax.axis_index("c")
    t = lax.axis_index("s")
    return c, t, c * NS + t


DCH = 200

@functools.partial(
    pl.kernel, mesh=_mesh(),
    out_type=jax.ShapeDtypeStruct((NC * NP, 128), jnp.float32),
    scratch_types=[pltpu.VMEM((DCH,), jnp.int32),
                   pltpu.VMEM((DCH, 128), jnp.float32),
                   pltpu.VMEM_SHARED((NP, 128), jnp.float32)],
)
def _k_deg(d_hbm, ones_hbm, zeros_hbm, out_hbm, didx_v, ones_v, acc_sh):
    c, t, w = _worker()
    pltpu.sync_copy(zeros_hbm.at[pl.ds(t * ROWS_T, ROWS_T)],
                    acc_sh.at[pl.ds(t * ROWS_T, ROWS_T)])
    pltpu.sync_copy(ones_hbm, ones_v)
    plsc.subcore_barrier()
    base = w * EPW

    def chunk(i, carry):
        pltpu.sync_copy(d_hbm.at[pl.ds(base + i * DCH, DCH)], didx_v)
        pltpu.sync_copy(ones_v, acc_sh.at[didx_v], add=True)
        return carry

    lax.fori_loop(0, EPW // DCH, chunk, 0)
    plsc.subcore_barrier()
    pltpu.sync_copy(acc_sh.at[pl.ds(t * ROWS_T, ROWS_T)],
                    out_hbm.at[pl.ds(c * NP + t * ROWS_T, ROWS_T)])


GCH = 200

@functools.partial(
    pl.kernel, mesh=_mesh(),
    out_type=jax.ShapeDtypeStruct((NC * NP, 128), jnp.float32),
    scratch_types=[pltpu.VMEM((GCH,), jnp.int32),
                   pltpu.VMEM((GCH,), jnp.int32),
                   pltpu.VMEM((GCH, 128), jnp.float32),
                   pltpu.VMEM_SHARED((NP, 128), jnp.float32),
                   pltpu.SemaphoreType.DMA],
)
def _k_gcn(s_hbm, d_hbm, xwd_hbm, zeros_hbm, out_hbm,
           sidx_v, didx_v, rows_v, acc_sh, sem):
    c, t, w = _worker()
    sl = pl.ds(t * ROWS_T, ROWS_T)
    pltpu.sync_copy(zeros_hbm.at[sl], acc_sh.at[sl])
    plsc.subcore_barrier()
    base = w * EPW

    def chunk(i, carry):
        pltpu.sync_copy(s_hbm.at[pl.ds(base + i * GCH, GCH)], sidx_v)
        pltpu.async_copy(xwd_hbm.at[sidx_v], rows_v, sem).wait()
        pltpu.sync_copy(d_hbm.at[pl.ds(base + i * GCH, GCH)], didx_v)
        pltpu.sync_copy(rows_v, acc_sh.at[didx_v], add=True)
        return carry

    lax.fori_loop(0, EPW // GCH, chunk, 0)
    plsc.subcore_barrier()
    pltpu.sync_copy(acc_sh.at[sl],
                    out_hbm.at[pl.ds(c * NP + t * ROWS_T, ROWS_T)])


def _make_gat(CE, SNP, HROWS, CH):
    EPT = CE // NS
    DRT = SNP // NS

    @functools.partial(
        pl.kernel, mesh=_mesh(),
        out_type=jax.ShapeDtypeStruct((NC * SNP, 128), jnp.float32),
        scratch_types=[pltpu.VMEM((CH,), jnp.int32),
                       pltpu.VMEM((CH,), jnp.int32),
                       pltpu.VMEM((CH,), jnp.int32),
                       pltpu.VMEM((CH, 128), jnp.float32),
                       pltpu.VMEM((CH, 128), jnp.float32),
                       pltpu.VMEM((CH, 128), jnp.float32),
                       pltpu.VMEM((CH, 128), jnp.float32),
                       pltpu.VMEM((CH, 128), jnp.float32),
                       pltpu.VMEM((16, 128), jnp.float32),
                       pltpu.VMEM_SHARED((SNP, 128), jnp.float32),
                       pltpu.VMEM_SHARED((SNP, 128), jnp.float32),
                       pltpu.SemaphoreType.DMA],
    )
    def k(s_hbm, dc_hbm, asb_hbm, adb_hbm, mb_hbm, h_hbm, z128_hbm,
          num_hbm,
          sidx_v, dcidx_v, aidx_v, asg_v, adg_v, exb_v,
          deng_v, hg_v, mb_v, den_sh, num_sh, sem):
        c, t, w = _worker()
        sl0 = pl.ds(t * DRT, DRT)
        pltpu.sync_copy(z128_hbm.at[sl0], den_sh.at[sl0])
        pltpu.sync_copy(z128_hbm.at[sl0], num_sh.at[sl0])
        ci = jnp.broadcast_to(c, (16,)).astype(jnp.int32)
        pltpu.async_copy(mb_hbm.at[ci], mb_v, sem).wait()
        plsc.subcore_barrier()
        mb = [mb_v[0, pl.ds(kk * 16, 16)] for kk in range(8)]
        base0 = t * EPT

        def adj_src(j, carry2):
            sl = pl.ds(j * 16, 16)
            aidx_v[sl] = sidx_v[sl] + c * HROWS
            return carry2

        def adj_dst(j, carry2):
            sl = pl.ds(j * 16, 16)
            aidx_v[sl] = dcidx_v[sl] + c * SNP
            return carry2

        def pass_a(i, carry):
            b = base0 + i * CH
            pltpu.sync_copy(s_hbm.at[pl.ds(b, CH)], sidx_v)
            pltpu.sync_copy(dc_hbm.at[pl.ds(b, CH)], dcidx_v)
            lax.fori_loop(0, CH // 16, adj_src, 0)
            pltpu.async_copy(asb_hbm.at[aidx_v], asg_v, sem).wait()
            lax.fori_loop(0, CH // 16, adj_dst, 0)
            pltpu.async_copy(adb_hbm.at[aidx_v], adg_v, sem).wait()

            def row(e, carry2):
                for kk in range(8):
                    csl = pl.ds(kk * 16, 16)
                    v = asg_v[e, csl] + adg_v[e, csl]
                    v = jnp.maximum(v, 0.2 * v)
                    exb_v[e, csl] = jnp.exp(v - mb[kk])
                return carry2

            lax.fori_loop(0, CH, row, 0)
            pltpu.sync_copy(exb_v, den_sh.at[dcidx_v], add=True)
            return carry

        lax.fori_loop(0, EPT // CH, pass_a, 0)
        plsc.subcore_barrier()

        def pass_b(i, carry):
            b = base0 + i * CH
            pltpu.sync_copy(s_hbm.at[pl.ds(b, CH)], sidx_v)
            pltpu.sync_copy(dc_hbm.at[pl.ds(b, CH)], dcidx_v)
            lax.fori_loop(0, CH // 16, adj_src, 0)
            pltpu.async_copy(h_hbm.at[aidx_v], hg_v, sem).wait()
            pltpu.async_copy(asb_hbm.at[aidx_v], asg_v, sem).wait()
            lax.fori_loop(0, CH // 16, adj_dst, 0)
            pltpu.async_copy(adb_hbm.at[aidx_v], adg_v, sem).wait()
            pltpu.sync_copy(den_sh.at[dcidx_v], deng_v)

            def row(e, carry2):
                for kk in range(8):
                    csl = pl.ds(kk * 16, 16)
                    v = asg_v[e, csl] + adg_v[e, csl]
                    v = jnp.maximum(v, 0.2 * v)
                    ex = jnp.exp(v - mb[kk])
                    al = ex / (deng_v[e, csl] + 1e-16)
                    hg_v[e, csl] = hg_v[e, csl] * al
                return carry2

            lax.fori_loop(0, CH, row, 0)
            pltpu.sync_copy(hg_v, num_sh.at[dcidx_v], add=True)
            return carry

        lax.fori_loop(0, EPT // CH, pass_b, 0)
        plsc.subcore_barrier()
        pltpu.sync_copy(num_sh.at[sl0],
                        num_hbm.at[pl.ds(c * SNP + t * DRT, DRT)])

    return k


_k_gat1 = _make_gat(C1, SNP1, NP, 64)
_k_gat2 = _make_gat(C2, SNP2, SNP1, 64)


def _core_rep(a):
    return jnp.concatenate(
        [jnp.repeat(a[:, :4], 32, axis=1), jnp.repeat(a[:, 4:], 32, axis=1)], 0)


def _expand_att(a):
    ch = a.shape[1]
    eye = jnp.eye(HEADS, dtype=a.dtype)
    return (a[:, :, None] * eye[:, None, :]).reshape(HEADS * ch, HEADS)


def kernel(x, edge_index, batch, y, W_gcn, b_gcn, W1, as1, ad1, b1g, W2, as2, ad2, b2g, Wl1, bl1, Wl2, bl2):
    src, dst = edge_index[0], edge_index[1]
    ones128 = jnp.ones((DCH, 128), jnp.float32)
    zeros128n = jnp.zeros((NP, 128), jnp.float32)

    deg_p = _k_deg(dst, ones128, zeros128n)
    deg = deg_p[:NP, 0] + deg_p[NP:, 0] + 1.0
    dinv = deg ** -0.5
    xw = x @ W_gcn
    xwp = jnp.zeros((NP, 32), jnp.float32).at[:N].set(xw)
    xwd = xwp * dinv[:, None]
    xwd128 = jnp.zeros((NP, 128), jnp.float32).at[:, :32].set(xwd)
    agg_p = _k_gcn(src, dst, xwd128, zeros128n)
    agg = agg_p[:NP, :32] + agg_p[NP:, :32] + xwd
    x1 = jax.nn.elu(dinv[:, None] * agg + b_gcn)

    idx = jnp.searchsorted(batch, jnp.arange(G, dtype=batch.dtype)).astype(jnp.int32)
    mark0 = jnp.zeros(N, bool).at[idx].set(True)
    sel2 = mark0[dst]
    e2 = jnp.nonzero(sel2, size=C2 - G, fill_value=E)[0]
    v2 = e2 < E
    e2c = jnp.minimum(e2, E - 1)
    s2o = jnp.where(v2, src[e2c], 0).astype(jnp.int32)
    d2o = jnp.where(v2, dst[e2c], N).astype(jnp.int32)
    s2o = jnp.concatenate([s2o, idx])
    d2o = jnp.concatenate([d2o, idx])
    mark1 = mark0.at[jnp.where(v2, s2o[:C2 - G], N)].set(True, mode="drop")
    sel1 = mark1[dst]
    e1 = jnp.nonzero(sel1, size=C1 - S1CAP, fill_value=E)[0]
    v1 = e1 < E
    e1c = jnp.minimum(e1, E - 1)
    s1o = jnp.where(v1, src[e1c], 0).astype(jnp.int32)
    d1o = jnp.where(v1, dst[e1c], N).astype(jnp.int32)
    n1 = jnp.nonzero(mark1, size=S1CAP, fill_value=N)[0].astype(jnp.int32)
    nv1 = n1 < N
    n1c = jnp.where(nv1, n1, 0)
    s1o = jnp.concatenate([s1o, n1c])
    d1o = jnp.concatenate([d1o, jnp.where(nv1, n1, N)])
    d1c = jnp.searchsorted(n1, d1o).astype(jnp.int32)
    s2c = jnp.searchsorted(n1, jnp.minimum(s2o, N - 1)).astype(jnp.int32)
    d2c = jnp.searchsorted(idx, d2o).astype(jnp.int32)

    h1 = x1 @ W1
    a_s1 = h1 @ _expand_att(as1)
    a_d1 = h1 @ _expand_att(ad1)
    M1 = a_s1.max(0) + a_d1.max(0)
    asb1 = _core_rep(a_s1)
    ad1t = jnp.zeros((SNP1, 8), jnp.float32).at[:S1CAP].set(a_d1[n1c])
    adb1 = _core_rep(ad1t)
    mb1 = _core_rep(M1[None, :])
    hcat1 = jnp.concatenate([h1[:, :128], h1[:, 128:]], 0)
    z128a = jnp.zeros((SNP1, 128), jnp.float32)
    num1 = _k_gat1(s1o, d1c, asb1, adb1, mb1, hcat1, z128a)
    x2c = jax.nn.elu(jnp.concatenate([num1[:SNP1], num1[SNP1:]], 1) + b1g)

    h2 = x2c @ W2
    a_s2 = h2 @ _expand_att(as2)
    a_d2 = h2 @ _expand_att(ad2)
    idx_c = jnp.searchsorted(n1, idx).astype(jnp.int32)
    ns1 = jnp.sum(nv1)
    vrow = (jnp.arange(SNP1) < ns1)[:, None]
    M2 = jnp.where(vrow, a_s2, -jnp.inf).max(0) + a_d2[idx_c].max(0)
    asb2 = _core_rep(a_s2)
    ad2t = jnp.zeros((SNP2, 8), jnp.float32).at[:G].set(a_d2[idx_c])
    adb2 = _core_rep(ad2t)
    mb2 = _core_rep(M2[None, :])
    hcat2 = jnp.concatenate([h2[:, :128], h2[:, 128:]], 0)
    z128b = jnp.zeros((SNP2, 128), jnp.float32)
    num2 = _k_gat2(s2c, d2c, asb2, adb2, mb2, hcat2, z128b)
    x3c = jax.nn.elu(jnp.concatenate([num2[:SNP2], num2[SNP2:]], 1) + b2g)

    pos = jnp.searchsorted(idx, idx).astype(jnp.int32)
    cat = jnp.concatenate([x1[idx], x2c[idx_c], x3c[pos]], axis=1)
    feat = cat @ Wl1 + bl1
    h = jax.nn.elu(feat)
    logits = h @ Wl2 + bl2
    logp = jax.nn.log_softmax(logits, axis=1)
    prob = jax.nn.softmax(logits, axis=1)
    loss = -jnp.mean(logp[jnp.arange(G), y])
    pred = jnp.argmax(logp, axis=1)
    return (logp, loss, pred, prob, feat)
